# baseline (speedup 1.0000x reference)
import functools

import jax
import jax.numpy as jnp
from jax import lax
from jax.experimental import pallas as pl
from jax.experimental.pallas import tpu as pltpu
from jax.experimental.pallas import tpu_sc as plsc

N = 10000
E = 320000
D = 128
H = 4
HD = D // H

NC = 2
NS = 16
NW = NC * NS
L = 16

C = 32
PER_W = 10240
E_PAD = NW * PER_W
NCH = PER_W // C
C2 = 128
NCH2 = PER_W // C2
SEGW = N * H + 960
N_PAD = 10240
ROWS_PER_TILE = N_PAD // NS
ZROWS = 128

import numpy as _np
_PP = _np.arange(128)
_PERM = (( _PP % 16) // 4) * 32 + ((_PP % 16) % 4) * 8 + (_PP // 16)
_INVPERM = _np.argsort(_PERM)

_mesh = plsc.VectorSubcoreMesh(core_axis_name="c", subcore_axis_name="s")
_sc_params = pltpu.CompilerParams(needs_layout_passes=False)


def _mish_attn(v, a):
    u = jnp.exp(jnp.minimum(v, 30.0))
    t = u * (u + 2.0)
    return v * (t / (t + 2.0)) * a


def _mm_body(x_ref, ws_ref, wr_ref, bs_ref, br_ref, xs_ref, xr_ref):
    xv = x_ref[...]
    xs_ref[...] = jnp.dot(xv, ws_ref[...], preferred_element_type=jnp.float32) + bs_ref[...]
    xr_ref[...] = jnp.dot(xv, wr_ref[...], preferred_element_type=jnp.float32) + br_ref[...]


def _node_transform(x, Wsm, Wrm, bs, br):
    return pl.pallas_call(
        _mm_body,
        grid=(10,),
        in_specs=[
            pl.BlockSpec((1000, D), lambda i: (i, 0)),
            pl.BlockSpec((D, D), lambda i: (0, 0)),
            pl.BlockSpec((D, D), lambda i: (0, 0)),
            pl.BlockSpec((1, D), lambda i: (0, 0)),
            pl.BlockSpec((1, D), lambda i: (0, 0)),
        ],
        out_specs=[
            pl.BlockSpec((1000, D), lambda i: (i, 0)),
            pl.BlockSpec((1000, D), lambda i: (i, 0)),
        ],
        out_shape=[
            jax.ShapeDtypeStruct((N, D), jnp.float32),
            jax.ShapeDtypeStruct((N, D), jnp.float32),
        ],
    )(x, Wsm, Wrm, bs, br)


def _pass1_body(xs_hbm, xr_hbm, send_hbm, recv_hbm, attn_hbm,
                w_hbm, outp_hbm,
                sidx_all, ridx_all, ridx_st, bufS4, bufR2, wbuf2, attnv, outacc,
                gS0, gS1, gS2, gS3, gR0, gR1, aw0, aw1, ss0, ss1, ss2, ss3):
    cid = lax.axis_index("c")
    sid = lax.axis_index("s")
    wid = sid * NC + cid
    gS = (gS0, gS1, gS2, gS3)
    gR = (gR0, gR1)
    aw = (aw0, aw1)
    ss = (ss0, ss1, ss2, ss3)

    pltpu.sync_copy(attn_hbm, attnv)
    pltpu.sync_copy(send_hbm.at[pl.ds(wid * PER_W, PER_W)], sidx_all)
    pltpu.sync_copy(recv_hbm.at[pl.ds(wid * PER_W, PER_W)], ridx_all)

    def zbody(i, carry):
        for j in range(D // L):
            bufS4[0, i, pl.ds(L * j, L)] = jnp.zeros((L,), jnp.float32)
        return carry
    lax.fori_loop(0, C, zbody, 0)
    for k in range(ROWS_PER_TILE // C):
        pltpu.sync_copy(bufS4.at[0],
                        outacc.at[pl.ds(sid * ROWS_PER_TILE + k * C, C)])
    plsc.subcore_barrier()

    iota = lax.iota(jnp.int32, L)
    av = [attnv[pl.ds(16 * j, L)] for j in range(8)]

    def gather_descs(g, p, q):
        dS = pltpu.make_async_copy(
            xs_hbm.at[sidx_all.at[pl.ds(g * C, C)]], bufS4.at[q], gS[q])
        dR = pltpu.make_async_copy(
            xr_hbm.at[ridx_all.at[pl.ds(g * C, C)]], bufR2.at[p], gR[p])
        return dS, dR

    def issue_gather(g, p, q):
        dS, dR = gather_descs(g, p, q)
        dS.start()
        dR.start()

    def sdesc(q):
        return pltpu.make_async_copy(bufS4.at[q], outacc.at[ridx_st.at[q]],
                                     ss[q])

    issue_gather(0, 0, 0)
    issue_gather(1, 1, 1)

    def half(g, p, q):
        pch = p * C * H
        dS, dR = gather_descs(g, p, q)
        dS.wait()
        dR.wait()

        @pl.when(g >= 2)
        def _():
            pltpu.make_async_copy(
                wbuf2.at[pl.ds(pch, C * H)], w_hbm.at[pl.ds(0, C * H)],
                aw[p]).wait()

        base = wid * PER_W + g * C

        def edge2(i, ecarry):
            for u in range(2):
                c = 2 * i + u
                svregs = []
                tsum = None
                for j in range(8):
                    s = bufS4[q, c, pl.ds(16 * j, L)]
                    r = bufR2[p, c, pl.ds(16 * j, L)]
                    m = _mish_attn(s + r, av[j])
                    svregs.append(s)
                    tsum = m if tsum is None else tsum + m
                cs = plsc.cumsum(tsum)
                csh = jnp.take_along_axis(cs, jnp.maximum(iota - 4, 0),
                                          axis=0)
                d = cs - jnp.where(iota >= 4, csh, 0.0)
                wv = jnp.exp(jnp.minimum(d, 60.0))
                wv = jnp.where(base + c < E, wv, 0.0)
                plsc.store_scatter(wbuf2, [pch + 4 * c + (iota >> 2)], wv,
                                   mask=(iota & 3) == 3)
                wexp = jnp.take_along_axis(wv, (iota >> 2) * 4 + 3, axis=0)
                for j in range(8):
                    bufS4[q, c, pl.ds(16 * j, L)] = svregs[j] * wexp
            return ecarry
        lax.fori_loop(0, C // 2, edge2, 0)

        pltpu.async_copy(wbuf2.at[pl.ds(pch, C * H)],
                         w_hbm.at[pl.ds(base * 4, C * H)], aw[p])

        for j in range(C // L):
            ridx_st[q, pl.ds(L * j, L)] = ridx_all[pl.ds(g * C + L * j, L)]

        pltpu.async_copy(bufS4.at[q], outacc.at[ridx_st.at[q]], ss[q],
                         add=True)

        q2 = (q + 2) % 4

        @pl.when(jnp.logical_and(g >= 2, g + 2 < NCH))
        def _():
            sdesc(q2).wait()

        @pl.when(g + 2 < NCH)
        def _():
            issue_gather(g + 2, p, q2)

    def kbody(k, carry):
        for u in range(4):
            half(4 * k + u, u % 2, u)
        return carry
    lax.fori_loop(0, NCH // 4, kbody, 0)

    for u in range(4):
        sdesc(u).wait()

    for p in range(2):
        pltpu.make_async_copy(
            wbuf2.at[pl.ds(p * C * H, C * H)], w_hbm.at[pl.ds(0, C * H)],
            aw[p]).wait()

    plsc.subcore_barrier()
    s0 = sid * ROWS_PER_TILE
    pltpu.sync_copy(outacc.at[pl.ds(s0, ROWS_PER_TILE)],
                    outp_hbm.at[cid, pl.ds(s0, ROWS_PER_TILE)])


def _pass1():
    return pl.kernel(
        _pass1_body,
        out_type=[
            jax.ShapeDtypeStruct((E_PAD * H,), jnp.float32),
            jax.ShapeDtypeStruct((NC, N_PAD, D), jnp.float32),
        ],
        mesh=_mesh,
        compiler_params=_sc_params,
        scratch_types=[
            pltpu.VMEM((PER_W,), jnp.int32),
            pltpu.VMEM((PER_W,), jnp.int32),
            pltpu.VMEM((4, C), jnp.int32),
            pltpu.VMEM((4, C, D), jnp.float32),
            pltpu.VMEM((2, C, D), jnp.float32),
            pltpu.VMEM((2 * C * H,), jnp.float32),
            pltpu.VMEM((D,), jnp.float32),
            pltpu.VMEM_SHARED((N_PAD, D), jnp.float32),
        ] + [pltpu.SemaphoreType.DMA] * 12,
    )


def _pass2_body(w_hbm, recv_hbm,
                segp_hbm,
                ridx_all, wbuf2, segsum, wl0, wl1):
    cid = lax.axis_index("c")
    sid = lax.axis_index("s")
    wid = sid * NC + cid
    wl = (wl0, wl1)

    pltpu.sync_copy(recv_hbm.at[pl.ds(wid * PER_W, PER_W)], ridx_all)

    def zbody(i, carry):
        segsum[pl.ds(i * L, L)] = jnp.zeros((L,), jnp.float32)
        return carry
    lax.fori_loop(0, SEGW // L, zbody, 0)

    iota = lax.iota(jnp.int32, L)
    lane_c = iota >> 2
    lane_h = iota & 3

    def wdesc(g, p):
        base4 = (wid * PER_W + g * C2) * 4
        return pltpu.make_async_copy(
            w_hbm.at[pl.ds(base4, C2 * H)],
            wbuf2.at[pl.ds(p * C2 * H, C2 * H)], wl[p])

    wdesc(0, 0).start()
    wdesc(1, 1).start()

    def half(g, p):
        pch = p * C2 * H
        wdesc(g, p).wait()
        for j in range(C2 * H // L):
            wv = wbuf2[pl.ds(pch + L * j, L)]
            cvec = lane_c + 4 * j
            rr = plsc.load_gather(ridx_all, [g * C2 + cvec])
            plsc.addupdate_scatter(segsum, [rr * 4 + lane_h], wv)

        @pl.when(g + 2 < NCH2)
        def _():
            wdesc(g + 2, p).start()

    def kbody(k, carry):
        half(2 * k, 0)
        half(2 * k + 1, 1)
        return carry
    lax.fori_loop(0, NCH2 // 2, kbody, 0)

    pltpu.sync_copy(segsum, segp_hbm.at[wid])


def _pass2():
    return pl.kernel(
        _pass2_body,
        out_type=jax.ShapeDtypeStruct((NW, SEGW), jnp.float32),
        mesh=_mesh,
        compiler_params=_sc_params,
        scratch_types=[
            pltpu.VMEM((PER_W,), jnp.int32),
            pltpu.VMEM((2 * C2 * H,), jnp.float32),
            pltpu.VMEM((SEGW,), jnp.float32),
            pltpu.SemaphoreType.DMA,
            pltpu.SemaphoreType.DMA,
        ],
    )


def _comb_body(p_ref, o_ref):
    o_ref[...] = jnp.sum(p_ref[...], axis=0)


def _combine(segp):
    return pl.pallas_call(
        _comb_body,
        out_shape=jax.ShapeDtypeStruct((SEGW // D, D), jnp.float32),
    )(segp.reshape(NW, SEGW // D, D))


def _add_body(p_ref, s_ref, o_ref):
    inv = 1.0 / jnp.maximum(s_ref[...], 1e-30)
    invx = jnp.tile(jnp.repeat(inv, 4, axis=1), (1, 8))
    o_ref[...] = (p_ref[0] + p_ref[1]) * invx


def _final_add(outp, seg4):
    return pl.pallas_call(
        _add_body,
        grid=(10,),
        in_specs=[
            pl.BlockSpec((NC, 1000, D), lambda i: (0, i, 0)),
            pl.BlockSpec((1000, H), lambda i: (i, 0)),
        ],
        out_specs=pl.BlockSpec((1000, D), lambda i: (i, 0)),
        out_shape=jax.ShapeDtypeStruct((N, D), jnp.float32),
    )(outp, seg4)


def kernel(x, edge_index, Ws_k, Ws_b, Wr_k, Wr_b, attn_k, attn_b):
    ei = edge_index.astype(jnp.int32)
    send = jnp.pad(ei[0], (0, E_PAD - E))
    recv = jnp.pad(ei[1], (0, E_PAD - E))
    Wsm = Ws_k.reshape(D, D)[:, _PERM]
    Wrm = Wr_k.reshape(D, D)[:, _PERM]
    bs = Ws_b.reshape(1, D)[:, _PERM]
    br = Wr_b.reshape(1, D)[:, _PERM]
    attn = attn_k.reshape(HD)[_PERM % HD]

    xs, xr = _node_transform(x, Wsm, Wrm, bs, br)
    w_flat, outp = _pass1()(xs, xr, send, recv, attn)
    segp = _pass2()(w_flat, recv)
    seg = _combine(segp)
    seg4 = seg.reshape(-1)[: N * H].reshape(N, H)
    return _final_add(outp, seg4)[:, _INVPERM]


# speedup vs baseline: 1.1353x; 1.1353x over previous
import functools

import jax
import jax.numpy as jnp
from jax import lax
from jax.experimental import pallas as pl
from jax.experimental.pallas import tpu as pltpu
from jax.experimental.pallas import tpu_sc as plsc

N = 10000
E = 320000
D = 128
H = 4
HD = D // H

NC = 2
NS = 16
NW = NC * NS
L = 16

C = 32
PER_W = 10240
E_PAD = NW * PER_W
NCH = PER_W // C
C2 = 128
NCH2 = PER_W // C2
SEGW = N * H + 960
N_PAD = 10240
ROWS_PER_TILE = N_PAD // NS
ZROWS = 128

import numpy as _np
_PP = _np.arange(128)
_PERM = (( _PP % 16) // 4) * 32 + ((_PP % 16) % 4) * 8 + (_PP // 16)
_INVPERM = _np.argsort(_PERM)

_mesh = plsc.VectorSubcoreMesh(core_axis_name="c", subcore_axis_name="s")
_sc_params = pltpu.CompilerParams(needs_layout_passes=False)


def _mish_attn(v, a):
    u = jnp.exp(jnp.minimum(v, 30.0))
    t = u * (u + 2.0)
    return v * (t / (t + 2.0)) * a


def _mm_body(x_ref, ws_ref, wr_ref, bs_ref, br_ref, xs_ref, xr_ref):
    xv = x_ref[...]
    xs_ref[...] = jnp.dot(xv, ws_ref[...], preferred_element_type=jnp.float32) + bs_ref[...]
    xr_ref[...] = jnp.dot(xv, wr_ref[...], preferred_element_type=jnp.float32) + br_ref[...]


def _node_transform(x, Wsm, Wrm, bs, br):
    return pl.pallas_call(
        _mm_body,
        grid=(10,),
        in_specs=[
            pl.BlockSpec((1000, D), lambda i: (i, 0)),
            pl.BlockSpec((D, D), lambda i: (0, 0)),
            pl.BlockSpec((D, D), lambda i: (0, 0)),
            pl.BlockSpec((1, D), lambda i: (0, 0)),
            pl.BlockSpec((1, D), lambda i: (0, 0)),
        ],
        out_specs=[
            pl.BlockSpec((1000, D), lambda i: (i, 0)),
            pl.BlockSpec((1000, D), lambda i: (i, 0)),
        ],
        out_shape=[
            jax.ShapeDtypeStruct((N, D), jnp.float32),
            jax.ShapeDtypeStruct((N, D), jnp.float32),
        ],
    )(x, Wsm, Wrm, bs, br)


def _pass1_body(xs_hbm, xr_hbm, send_hbm, recv_hbm, attn_hbm,
                w_hbm, outp_hbm,
                sidx_all, ridx_all, ridx_st, bufS4, bufR2, wbuf2, attnv, outacc,
                gS0, gS1, gS2, gS3, gR0, gR1, aw0, aw1, ss0, ss1, ss2, ss3):
    cid = lax.axis_index("c")
    sid = lax.axis_index("s")
    wid = sid * NC + cid
    gS = (gS0, gS1, gS2, gS3)
    gR = (gR0, gR1)
    aw = (aw0, aw1)
    ss = (ss0, ss1, ss2, ss3)

    pltpu.sync_copy(attn_hbm, attnv)
    pltpu.sync_copy(send_hbm.at[pl.ds(wid * PER_W, PER_W)], sidx_all)
    pltpu.sync_copy(recv_hbm.at[pl.ds(wid * PER_W, PER_W)], ridx_all)

    def zbody(i, carry):
        for j in range(D // L):
            bufS4[0, i, pl.ds(L * j, L)] = jnp.zeros((L,), jnp.float32)
        return carry
    lax.fori_loop(0, C, zbody, 0)
    for k in range(ROWS_PER_TILE // C):
        pltpu.sync_copy(bufS4.at[0],
                        outacc.at[pl.ds(sid * ROWS_PER_TILE + k * C, C)])
    plsc.subcore_barrier()

    iota = lax.iota(jnp.int32, L)
    lane_c = iota >> 2
    a0 = attnv[pl.ds(0, L)]
    a1 = attnv[pl.ds(L, L)]

    def gather_descs(g, p, q):
        dS = pltpu.make_async_copy(
            xs_hbm.at[sidx_all.at[pl.ds(g * C, C)]], bufS4.at[q], gS[q])
        dR = pltpu.make_async_copy(
            xr_hbm.at[ridx_all.at[pl.ds(g * C, C)]], bufR2.at[p], gR[p])
        return dS, dR

    def issue_gather(g, p, q):
        dS, dR = gather_descs(g, p, q)
        dS.start()
        dR.start()

    def sdesc(q):
        return pltpu.make_async_copy(bufS4.at[q], outacc.at[ridx_st.at[q]],
                                     ss[q])

    issue_gather(0, 0, 0)
    issue_gather(1, 1, 1)

    def half(g, p, q):
        pch = p * C * H
        dS, dR = gather_descs(g, p, q)
        dS.wait()
        dR.wait()

        @pl.when(g >= 2)
        def _():
            pltpu.make_async_copy(
                wbuf2.at[pl.ds(pch, C * H)], w_hbm.at[pl.ds(0, C * H)],
                aw[p]).wait()

        def edge2(i, ecarry):
            for u in range(2):
                c = 2 * i + u
                ps = []
                for j in range(8):
                    s = bufS4[q, c, pl.ds(16 * j, L)]
                    r = bufR2[p, c, pl.ds(16 * j, L)]
                    ps.append(_mish_attn(s + r, a0 if j % 2 == 0 else a1))
                acc = jnp.zeros((L,), jnp.float32)
                for h in range(4):
                    sh = jnp.sum(ps[2 * h] + ps[2 * h + 1])
                    acc = jnp.where(iota == h, sh, acc)
                plsc.store_scatter(wbuf2, [pch + 4 * c + iota], acc,
                                   mask=iota < 4)
            return ecarry
        lax.fori_loop(0, C // 2, edge2, 0)

        base = wid * PER_W + g * C
        for j in range(C * H // L):
            wv = wbuf2[pl.ds(pch + L * j, L)]
            cvec = lane_c + 4 * j
            valid = (cvec + base) < E
            wv = jnp.exp(jnp.minimum(wv, 60.0))
            wv = jnp.where(valid, wv, 0.0)
            wbuf2[pl.ds(pch + L * j, L)] = wv

        pltpu.async_copy(wbuf2.at[pl.ds(pch, C * H)],
                         w_hbm.at[pl.ds(base * 4, C * H)], aw[p])

        for j in range(C // L):
            ridx_st[q, pl.ds(L * j, L)] = ridx_all[pl.ds(g * C + L * j, L)]

        def edge4(e4, ecarry):
            cv = wbuf2[pl.ds(pch + L * e4, L)]
            for k in range(4):
                c = 4 * e4 + k
                for jj in range(8):
                    ch = cv[4 * k + jj // 2]
                    bufS4[q, c, pl.ds(16 * jj, L)] = (
                        bufS4[q, c, pl.ds(16 * jj, L)] * ch)
            return ecarry
        lax.fori_loop(0, C * H // L, edge4, 0)

        pltpu.async_copy(bufS4.at[q], outacc.at[ridx_st.at[q]], ss[q],
                         add=True)

        q2 = (q + 2) % 4

        @pl.when(jnp.logical_and(g >= 2, g + 2 < NCH))
        def _():
            sdesc(q2).wait()

        @pl.when(g + 2 < NCH)
        def _():
            issue_gather(g + 2, p, q2)

    def kbody(k, carry):
        for u in range(4):
            half(4 * k + u, u % 2, u)
        return carry
    lax.fori_loop(0, NCH // 4, kbody, 0)

    for u in range(4):
        sdesc(u).wait()

    for p in range(2):
        pltpu.make_async_copy(
            wbuf2.at[pl.ds(p * C * H, C * H)], w_hbm.at[pl.ds(0, C * H)],
            aw[p]).wait()

    plsc.subcore_barrier()
    s0 = sid * ROWS_PER_TILE
    pltpu.sync_copy(outacc.at[pl.ds(s0, ROWS_PER_TILE)],
                    outp_hbm.at[cid, pl.ds(s0, ROWS_PER_TILE)])


def _pass1():
    return pl.kernel(
        _pass1_body,
        out_type=[
            jax.ShapeDtypeStruct((E_PAD * H,), jnp.float32),
            jax.ShapeDtypeStruct((NC, N_PAD, D), jnp.float32),
        ],
        mesh=_mesh,
        compiler_params=_sc_params,
        scratch_types=[
            pltpu.VMEM((PER_W,), jnp.int32),
            pltpu.VMEM((PER_W,), jnp.int32),
            pltpu.VMEM((4, C), jnp.int32),
            pltpu.VMEM((4, C, D), jnp.float32),
            pltpu.VMEM((2, C, D), jnp.float32),
            pltpu.VMEM((2 * C * H,), jnp.float32),
            pltpu.VMEM((HD,), jnp.float32),
            pltpu.VMEM_SHARED((N_PAD, D), jnp.float32),
        ] + [pltpu.SemaphoreType.DMA] * 12,
    )


def _pass2_body(w_hbm, recv_hbm,
                segp_hbm,
                ridx_all, wbuf2, segsum, wl0, wl1):
    cid = lax.axis_index("c")
    sid = lax.axis_index("s")
    wid = sid * NC + cid
    wl = (wl0, wl1)

    pltpu.sync_copy(recv_hbm.at[pl.ds(wid * PER_W, PER_W)], ridx_all)

    def zbody(i, carry):
        segsum[pl.ds(i * L, L)] = jnp.zeros((L,), jnp.float32)
        return carry
    lax.fori_loop(0, SEGW // L, zbody, 0)

    iota = lax.iota(jnp.int32, L)
    lane_c = iota >> 2
    lane_h = iota & 3

    def wdesc(g, p):
        base4 = (wid * PER_W + g * C2) * 4
        return pltpu.make_async_copy(
            w_hbm.at[pl.ds(base4, C2 * H)],
            wbuf2.at[pl.ds(p * C2 * H, C2 * H)], wl[p])

    wdesc(0, 0).start()
    wdesc(1, 1).start()

    def half(g, p):
        pch = p * C2 * H
        wdesc(g, p).wait()
        for j in range(C2 * H // L):
            wv = wbuf2[pl.ds(pch + L * j, L)]
            cvec = lane_c + 4 * j
            rr = plsc.load_gather(ridx_all, [g * C2 + cvec])
            plsc.addupdate_scatter(segsum, [rr * 4 + lane_h], wv)

        @pl.when(g + 2 < NCH2)
        def _():
            wdesc(g + 2, p).start()

    def kbody(k, carry):
        half(2 * k, 0)
        half(2 * k + 1, 1)
        return carry
    lax.fori_loop(0, NCH2 // 2, kbody, 0)

    pltpu.sync_copy(segsum, segp_hbm.at[wid])


def _pass2():
    return pl.kernel(
        _pass2_body,
        out_type=jax.ShapeDtypeStruct((NW, SEGW), jnp.float32),
        mesh=_mesh,
        compiler_params=_sc_params,
        scratch_types=[
            pltpu.VMEM((PER_W,), jnp.int32),
            pltpu.VMEM((2 * C2 * H,), jnp.float32),
            pltpu.VMEM((SEGW,), jnp.float32),
            pltpu.SemaphoreType.DMA,
            pltpu.SemaphoreType.DMA,
        ],
    )


def _comb_body(p_ref, o_ref):
    o_ref[...] = jnp.sum(p_ref[...], axis=0)


def _combine(segp):
    return pl.pallas_call(
        _comb_body,
        out_shape=jax.ShapeDtypeStruct((SEGW // D, D), jnp.float32),
    )(segp.reshape(NW, SEGW // D, D))


def _add_body(p_ref, s_ref, o_ref):
    inv = 1.0 / jnp.maximum(s_ref[...], 1e-30)
    invx = jnp.repeat(inv, HD, axis=1)
    o_ref[...] = (p_ref[0] + p_ref[1]) * invx


def _final_add(outp, seg4):
    return pl.pallas_call(
        _add_body,
        grid=(10,),
        in_specs=[
            pl.BlockSpec((NC, 1000, D), lambda i: (0, i, 0)),
            pl.BlockSpec((1000, H), lambda i: (i, 0)),
        ],
        out_specs=pl.BlockSpec((1000, D), lambda i: (i, 0)),
        out_shape=jax.ShapeDtypeStruct((N, D), jnp.float32),
    )(outp, seg4)


def kernel(x, edge_index, Ws_k, Ws_b, Wr_k, Wr_b, attn_k, attn_b):
    ei = edge_index.astype(jnp.int32)
    send = jnp.pad(ei[0], (0, E_PAD - E))
    recv = jnp.pad(ei[1], (0, E_PAD - E))
    Wsm = Ws_k.reshape(D, D)
    Wrm = Wr_k.reshape(D, D)
    bs = Ws_b.reshape(1, D)
    br = Wr_b.reshape(1, D)
    attn = attn_k.reshape(HD)

    xs, xr = _node_transform(x, Wsm, Wrm, bs, br)
    w_flat, outp = _pass1()(xs, xr, send, recv, attn)
    segp = _pass2()(w_flat, recv)
    seg = _combine(segp)
    seg4 = seg.reshape(-1)[: N * H].reshape(N, H)
    return _final_add(outp, seg4)
